# Initial kernel scaffold; baseline (speedup 1.0000x reference)
import jax
import jax.numpy as jnp
from jax.experimental import pallas as pl


def kernel(x, gate_w, w1, w3, w2):
    raise NotImplementedError("write your pallas kernel here")


# speedup vs baseline: 1.4194x; 1.4194x over previous
import functools

import jax
import jax.numpy as jnp
from jax.experimental import pallas as pl
from jax.experimental.pallas import tpu as pltpu

D_MODEL = 1024
D_FF = 2816
N_EXPERTS = 8

BM = 1024
BF = 256
NF = D_FF // BF


def _moe_dense_body(x_ref, gw_ref, w1_ref, w3_ref, w2_ref, out_ref, comb_ref):
    e = pl.program_id(1)
    f = pl.program_id(2)

    @pl.when(jnp.logical_and(e == 0, f == 0))
    def _router():
        xb = x_ref[...]
        logits = jax.lax.dot_general(
            xb, gw_ref[...], (((1,), (1,)), ((), ())),
            preferred_element_type=jnp.float32)
        col = jax.lax.broadcasted_iota(jnp.int32, (BM, N_EXPERTS), 1)
        m1 = jnp.max(logits, axis=1, keepdims=True)
        i1 = jnp.min(jnp.where(logits == m1, col, N_EXPERTS), axis=1,
                     keepdims=True)
        masked = jnp.where(col == i1, -jnp.inf, logits)
        m2 = jnp.max(masked, axis=1, keepdims=True)
        i2 = jnp.min(jnp.where(masked == m2, col, N_EXPERTS), axis=1,
                     keepdims=True)
        wa = 1.0 / (1.0 + jnp.exp(m2 - m1))
        wb = 1.0 - wa
        comb_ref[...] = (jnp.where(col == i1, wa, 0.0)
                         + jnp.where(col == i2, wb, 0.0))

    xb = x_ref[...]
    h1 = jax.lax.dot_general(xb, w1_ref[0], (((1,), (1,)), ((), ())),
                             preferred_element_type=jnp.float32)
    h3 = jax.lax.dot_general(xb, w3_ref[0], (((1,), (1,)), ((), ())),
                             preferred_element_type=jnp.float32)
    h = (h1 * jax.lax.logistic(h1)) * h3
    contrib = jax.lax.dot_general(h, w2_ref[0], (((1,), (1,)), ((), ())),
                                  preferred_element_type=jnp.float32)
    col2 = jax.lax.broadcasted_iota(jnp.int32, (BM, N_EXPERTS), 1)
    c_e = jnp.sum(jnp.where(col2 == e, comb_ref[...], 0.0), axis=1,
                  keepdims=True)
    contrib = contrib * c_e

    @pl.when(jnp.logical_and(e == 0, f == 0))
    def _init():
        out_ref[...] = contrib

    @pl.when(jnp.logical_not(jnp.logical_and(e == 0, f == 0)))
    def _acc():
        out_ref[...] += contrib


@functools.partial(jax.jit, static_argnames=("interpret",))
def _moe_dense(xr, gate_w, w1, w3, w2, interpret=False):
    T = xr.shape[0]
    nm = T // BM
    return pl.pallas_call(
        _moe_dense_body,
        grid=(nm, N_EXPERTS, NF),
        in_specs=[
            pl.BlockSpec((BM, D_MODEL), lambda m, e, f: (m, 0)),
            pl.BlockSpec((N_EXPERTS, D_MODEL), lambda m, e, f: (0, 0)),
            pl.BlockSpec((1, BF, D_MODEL), lambda m, e, f: (e, f, 0)),
            pl.BlockSpec((1, BF, D_MODEL), lambda m, e, f: (e, f, 0)),
            pl.BlockSpec((1, D_MODEL, BF), lambda m, e, f: (e, 0, f)),
        ],
        out_specs=pl.BlockSpec((BM, D_MODEL), lambda m, e, f: (m, 0)),
        out_shape=jax.ShapeDtypeStruct((T, D_MODEL), jnp.float32),
        scratch_shapes=[pltpu.VMEM((BM, N_EXPERTS), jnp.float32)],
        compiler_params=pltpu.CompilerParams(
            dimension_semantics=("arbitrary", "arbitrary", "arbitrary")),
        interpret=interpret,
    )(xr, gate_w, w1, w3, w2)


def kernel(x, gate_w, w1, w3, w2):
    B, S, D = x.shape
    xr = x.reshape(-1, D)
    out = _moe_dense(xr, gate_w, w1, w3, w2)
    return out.reshape(B, S, D)


# speedup vs baseline: 1.6116x; 1.1354x over previous
import dataclasses
import functools

import jax
import jax.numpy as jnp
from jax import lax
from jax.experimental import pallas as pl
from jax.experimental.pallas import tpu as pltpu
from jax.experimental.pallas import tpu_sc as plsc

D_MODEL = 1024
D_FF = 2816
N_EXPERTS = 8
T_TOKENS = 4096
N_ASSIGN = 2 * T_TOKENS

BMM = 512
MAXB = N_ASSIGN // BMM + N_EXPERTS
PADT = MAXB * BMM
BF = 256
NF = D_FF // BF

RCHUNK = 512
NRC = T_TOKENS // RCHUNK

NWORK = 32
RPW = PADT // NWORK
APW = N_ASSIGN // NWORK
TPW = T_TOKENS // NWORK
GR = 64
NT = 16


def _sc_compiler_params():
    cp = pltpu.CompilerParams()
    if "needs_layout_passes" in pltpu.CompilerParams.__dataclass_fields__:
        cp = dataclasses.replace(cp, needs_layout_passes=False)
    return cp


def _router_body(x_ref, gw_ref, pos_ref, wv_ref, be_ref, mb_ref,
                 oh0_ref, oh1_ref, run_ref):
    c = pl.program_id(0)

    @pl.when(c == 0)
    def _init():
        run_ref[...] = jnp.zeros((1, N_EXPERTS), jnp.float32)

    @pl.when(c < NRC)
    def _chunk():
        xb = x_ref[...]
        logits = lax.dot_general(xb, gw_ref[...], (((1,), (1,)), ((), ())),
                                 preferred_element_type=jnp.float32)
        col = lax.broadcasted_iota(jnp.int32, (RCHUNK, N_EXPERTS), 1)
        m1 = jnp.max(logits, axis=1, keepdims=True)
        i1 = jnp.min(jnp.where(logits == m1, col, N_EXPERTS), axis=1,
                     keepdims=True)
        masked = jnp.where(col == i1, -jnp.inf, logits)
        m2 = jnp.max(masked, axis=1, keepdims=True)
        i2 = jnp.min(jnp.where(masked == m2, col, N_EXPERTS), axis=1,
                     keepdims=True)
        wa = 1.0 / (1.0 + jnp.exp(m2 - m1))
        wb = 1.0 - wa

        oh_a = (col == i1).astype(jnp.float32)
        oh_b = (col == i2).astype(jnp.float32)
        rows = pl.ds(c * RCHUNK, RCHUNK)
        oh0_ref[rows, :] = oh_a
        oh1_ref[rows, :] = oh_b
        wv_ref[rows, :] = jnp.concatenate([wa, wb], axis=1)

        oh = jnp.concatenate([oh_a, oh_b], axis=0)
        n2 = 2 * RCHUNK
        ri = lax.broadcasted_iota(jnp.int32, (n2, n2), 0)
        ci = lax.broadcasted_iota(jnp.int32, (n2, n2), 1)
        tril = (ci < ri).astype(jnp.float32)
        excl = lax.dot_general(tril, oh, (((1,), (0,)), ((), ())),
                               preferred_element_type=jnp.float32)
        rank_local = jnp.sum(excl * oh, axis=1, keepdims=True)
        rank_glob = rank_local + jnp.sum(run_ref[...] * oh, axis=1,
                                         keepdims=True)
        pos_ref[rows, :] = jnp.concatenate(
            [rank_glob[0:RCHUNK], rank_glob[RCHUNK:n2]], axis=1
        ).astype(jnp.int32)
        run_ref[...] += jnp.sum(oh, axis=0, keepdims=True)

    @pl.when(c == NRC)
    def _finalize():
        counts = run_ref[...]
        pc = jnp.ceil(counts / BMM) * BMM
        ui = lax.broadcasted_iota(jnp.int32, (N_EXPERTS, N_EXPERTS), 0)
        uj = lax.broadcasted_iota(jnp.int32, (N_EXPERTS, N_EXPERTS), 1)
        upper = (ui < uj).astype(jnp.float32)
        pad_off = lax.dot_general(pc, upper, (((1,), (0,)), ((), ())),
                                  preferred_element_type=jnp.float32)
        offs0 = lax.dot_general(oh0_ref[...], pad_off,
                                (((1,), (1,)), ((), ())),
                                preferred_element_type=jnp.float32)
        offs1 = lax.dot_general(oh1_ref[...], pad_off,
                                (((1,), (1,)), ((), ())),
                                preferred_element_type=jnp.float32)
        pos_ref[...] += jnp.concatenate([offs0, offs1], axis=1).astype(
            jnp.int32)

        total = jnp.sum(pc, axis=1, keepdims=True)
        nact = (total / BMM).astype(jnp.int32)
        b_iota = lax.broadcasted_iota(jnp.int32, (MAXB, 1), 0)
        b_eff = jnp.minimum(b_iota, nact - 1)
        bc = jnp.broadcast_to(pad_off, (MAXB, N_EXPERTS))
        cmp = (b_eff.astype(jnp.float32) * BMM >= bc).astype(jnp.float32)
        be = jnp.sum(cmp, axis=1, keepdims=True).astype(jnp.int32) - 1
        be_ref[...] = be
        mb_ref[...] = b_eff


@functools.partial(jax.jit, static_argnames=("interpret",))
def _router(xr, gate_w, interpret=False):
    return pl.pallas_call(
        _router_body,
        grid=(NRC + 1,),
        in_specs=[
            pl.BlockSpec((RCHUNK, D_MODEL),
                         lambda c: (jnp.minimum(c, NRC - 1), 0)),
            pl.BlockSpec((N_EXPERTS, D_MODEL), lambda c: (0, 0)),
        ],
        out_specs=[
            pl.BlockSpec((T_TOKENS, 2), lambda c: (0, 0)),
            pl.BlockSpec((T_TOKENS, 2), lambda c: (0, 0)),
            pl.BlockSpec((MAXB, 1), lambda c: (0, 0)),
            pl.BlockSpec((MAXB, 1), lambda c: (0, 0)),
        ],
        out_shape=[
            jax.ShapeDtypeStruct((T_TOKENS, 2), jnp.int32),
            jax.ShapeDtypeStruct((T_TOKENS, 2), jnp.float32),
            jax.ShapeDtypeStruct((MAXB, 1), jnp.int32),
            jax.ShapeDtypeStruct((MAXB, 1), jnp.int32),
        ],
        scratch_shapes=[
            pltpu.VMEM((T_TOKENS, N_EXPERTS), jnp.float32),
            pltpu.VMEM((T_TOKENS, N_EXPERTS), jnp.float32),
            pltpu.VMEM((1, N_EXPERTS), jnp.float32),
        ],
        compiler_params=pltpu.CompilerParams(
            dimension_semantics=("arbitrary",)),
        interpret=interpret,
    )(xr, gate_w)


def _sc_gather(pos_flat, xr):
    mesh = plsc.VectorSubcoreMesh(core_axis_name="c", subcore_axis_name="s")

    @functools.partial(
        pl.kernel, mesh=mesh,
        out_type=jax.ShapeDtypeStruct((PADT, D_MODEL), jnp.float32),
        scratch_types=[
            pltpu.VMEM((N_ASSIGN,), jnp.int32),
            pltpu.VMEM((RPW,), jnp.int32),
            pltpu.VMEM((GR, D_MODEL), jnp.float32),
            pltpu.SemaphoreType.DMA,
        ],
        compiler_params=_sc_compiler_params(),
    )
    def k(pos_hbm, x_hbm, xs_hbm, pos_v, src_v, buf, sem):
        cid = lax.axis_index("c")
        sid = lax.axis_index("s")
        w = cid * 16 + sid
        lo = w * RPW
        pltpu.sync_copy(pos_hbm, pos_v)

        @pl.loop(0, RPW, step=16)
        def _zero(i):
            src_v[pl.ds(i, 16)] = jnp.zeros((16,), jnp.int32)

        lane = jnp.arange(16, dtype=jnp.int32)

        @pl.loop(0, N_ASSIGN, step=16)
        def _scatter(a):
            p = pos_v[pl.ds(a, 16)]
            tok = (a + lane) >> 1
            rel = p - lo
            m = jnp.logical_and(rel >= 0, rel < RPW)
            relc = jnp.clip(rel, 0, RPW - 1)
            plsc.store_scatter(src_v, [relc], tok, mask=m)

        @pl.loop(0, RPW, step=GR)
        def _gather(r):
            pltpu.async_copy(x_hbm.at[src_v.at[pl.ds(r, GR)]], buf,
                             sem).wait()
            pltpu.sync_copy(buf, xs_hbm.at[pl.ds(lo + r, GR)])

    return k(pos_flat, xr)


def _gmm_body(be_ref, mb_ref, xs_ref, w1_ref, w3_ref, w2_ref, out_ref):
    m = pl.program_id(0)
    f = pl.program_id(1)
    active = m == mb_ref[m, 0]

    @pl.when(active)
    def _compute():
        xb = xs_ref[...]
        h1 = lax.dot_general(xb, w1_ref[0], (((1,), (1,)), ((), ())),
                             preferred_element_type=jnp.float32)
        h3 = lax.dot_general(xb, w3_ref[0], (((1,), (1,)), ((), ())),
                             preferred_element_type=jnp.float32)
        h = (h1 * lax.logistic(h1)) * h3
        contrib = lax.dot_general(h, w2_ref[0], (((1,), (1,)), ((), ())),
                                  preferred_element_type=jnp.float32)

        @pl.when(f == 0)
        def _set():
            out_ref[...] = contrib

        @pl.when(f > 0)
        def _acc():
            out_ref[...] += contrib


@functools.partial(jax.jit, static_argnames=("interpret",))
def _gmm(be, mb, xs, w1, w3, w2, interpret=False):
    def f_eff(m, f, be_r, mb_r):
        return jnp.where(m == mb_r[m, 0], f, 0)

    grid_spec = pltpu.PrefetchScalarGridSpec(
        num_scalar_prefetch=2,
        grid=(MAXB, NF),
        in_specs=[
            pl.BlockSpec((BMM, D_MODEL),
                         lambda m, f, be_r, mb_r: (mb_r[m, 0], 0)),
            pl.BlockSpec((1, BF, D_MODEL),
                         lambda m, f, be_r, mb_r: (be_r[m, 0],
                                                   f_eff(m, f, be_r, mb_r),
                                                   0)),
            pl.BlockSpec((1, BF, D_MODEL),
                         lambda m, f, be_r, mb_r: (be_r[m, 0],
                                                   f_eff(m, f, be_r, mb_r),
                                                   0)),
            pl.BlockSpec((1, D_MODEL, BF),
                         lambda m, f, be_r, mb_r: (be_r[m, 0], 0,
                                                   f_eff(m, f, be_r, mb_r))),
        ],
        out_specs=pl.BlockSpec((BMM, D_MODEL),
                               lambda m, f, be_r, mb_r: (mb_r[m, 0], 0)),
    )
    return pl.pallas_call(
        _gmm_body,
        grid_spec=grid_spec,
        out_shape=jax.ShapeDtypeStruct((PADT, D_MODEL), jnp.float32),
        compiler_params=pltpu.CompilerParams(
            dimension_semantics=("arbitrary", "arbitrary")),
        interpret=interpret,
    )(be, mb, xs, w1, w3, w2)


def _sc_combine(y, pos_flat, wv_flat):
    mesh = plsc.VectorSubcoreMesh(core_axis_name="c", subcore_axis_name="s")

    @functools.partial(
        pl.kernel, mesh=mesh,
        out_type=jax.ShapeDtypeStruct((T_TOKENS, D_MODEL), jnp.float32),
        scratch_types=[
            pltpu.VMEM((APW,), jnp.int32),
            pltpu.VMEM((APW,), jnp.float32),
            pltpu.VMEM((2 * NT, D_MODEL), jnp.float32),
            pltpu.VMEM((NT, D_MODEL), jnp.float32),
            pltpu.SemaphoreType.DMA,
        ],
        compiler_params=_sc_compiler_params(),
    )
    def k(y_hbm, pos_hbm, wv_hbm, out_hbm, pos_v, wv_v, buf, obuf, sem):
        cid = lax.axis_index("c")
        sid = lax.axis_index("s")
        w = cid * 16 + sid
        t0 = w * TPW
        a0 = w * APW
        pltpu.sync_copy(pos_hbm.at[pl.ds(a0, APW)], pos_v)
        pltpu.sync_copy(wv_hbm.at[pl.ds(a0, APW)], wv_v)

        @pl.loop(0, TPW, step=NT)
        def _sub(t):
            pltpu.async_copy(y_hbm.at[pos_v.at[pl.ds(2 * t, 2 * NT)]], buf,
                             sem).wait()

            @pl.loop(0, NT)
            def _row(i):
                wa = plsc.load_gather(
                    wv_v, [jnp.full((16,), 2 * (t + i), jnp.int32)])
                wb = plsc.load_gather(
                    wv_v, [jnp.full((16,), 2 * (t + i) + 1, jnp.int32)])

                @pl.loop(0, D_MODEL, step=16)
                def _lane(j):
                    va = buf[2 * i, pl.ds(j, 16)]
                    vb = buf[2 * i + 1, pl.ds(j, 16)]
                    obuf[i, pl.ds(j, 16)] = wa * va + wb * vb

            pltpu.sync_copy(obuf, out_hbm.at[pl.ds(t0 + t, NT)])

    return k(y, pos_flat, wv_flat)


def kernel(x, gate_w, w1, w3, w2):
    B, S, D = x.shape
    xr = x.reshape(-1, D)
    pos, wv, be, mb = _router(xr, gate_w)
    pos_flat = pos.reshape(-1)
    wv_flat = wv.reshape(-1)
    xs = _sc_gather(pos_flat, xr)
    y = _gmm(be, mb, xs, w1, w3, w2)
    out = _sc_combine(y, pos_flat, wv_flat)
    return out.reshape(B, S, D)
